# Initial kernel scaffold; baseline (speedup 1.0000x reference)
import jax
import jax.numpy as jnp
from jax.experimental import pallas as pl


def kernel(points_to_interp, grid_values):
    raise NotImplementedError("write your pallas kernel here")


# speedup vs baseline: 43.7421x; 43.7421x over previous
import functools

import jax
import jax.numpy as jnp
from jax import lax
from jax.experimental import pallas as pl
from jax.experimental.pallas import tpu as pltpu
from jax.experimental.pallas import tpu_sc as plsc

F = 64
TICKS = 40
NC, NS, L = 2, 16, 16
NW = NC * NS
C = 128


def _interp_sc(n_points):
    pts_per_w = n_points // NW
    n_chunks = pts_per_w // C
    vecs = C // L

    mesh = plsc.VectorSubcoreMesh(core_axis_name="c", subcore_axis_name="s")

    @functools.partial(
        pl.kernel,
        mesh=mesh,
        out_type=jax.ShapeDtypeStruct((n_points, F), jnp.float32),
        scratch_types=[
            pltpu.VMEM((128,), jnp.float32),
            pltpu.VMEM((C,), jnp.float32),
            pltpu.VMEM((C,), jnp.float32),
            pltpu.VMEM((C,), jnp.float32),
            pltpu.VMEM((4, C), jnp.int32),
            pltpu.VMEM((8, C), jnp.float32),
            pltpu.VMEM((4, C, 2 * F), jnp.float32),
            pltpu.VMEM((C, F), jnp.float32),
            pltpu.SemaphoreType.DMA,
        ],
    )
    def body(ticks_hbm, xs_hbm, ys_hbm, zs_hbm, table_hbm, out_hbm,
             ticks_v, x_v, y_v, z_v, idx_v, w_v, rows_v, out_v, sem):
        wid = lax.axis_index("s") * NC + lax.axis_index("c")
        base = wid * pts_per_w
        pltpu.sync_copy(ticks_hbm, ticks_v)

        def axis_calc(x, tregs):
            def lookup(k):
                v0 = tregs[0].at[jnp.clip(k, 0, L - 1)].get(
                    mode="promise_in_bounds")
                v1 = tregs[1].at[jnp.clip(k - L, 0, L - 1)].get(
                    mode="promise_in_bounds")
                v2 = tregs[2].at[jnp.clip(k - 2 * L, 0, L - 1)].get(
                    mode="promise_in_bounds")
                return jnp.where(k < L, v0, jnp.where(k < 2 * L, v1, v2))

            est = ((x + 1.0) * 20.0).astype(jnp.int32)
            km1 = est - 1
            kp1 = est + 1

            def contrib(k):
                t = lookup(jnp.clip(k, 0, TICKS - 1))
                c = jnp.where(t < x, 1, 0)
                return jnp.where(k < 0, 1, jnp.where(k > TICKS - 1, 0, c))

            cnt = km1 + contrib(km1) + contrib(est) + contrib(kp1)
            ir = jnp.minimum(cnt, TICKS - 1)
            il = jnp.maximum(ir - 1, 0)
            tl = lookup(il)
            tr = lookup(ir)
            dl = jnp.maximum(x - tl, 0.0)
            dr = jnp.maximum(tr - x, 0.0)
            bz = (dl == 0.0) & (dr == 0.0)
            dl = jnp.where(bz, 1.0, dl)
            dr = jnp.where(bz, 1.0, dr)
            return il, ir, dl, dr

        def chunk_body(g, _):
            pt0 = base + g * C
            pltpu.sync_copy(xs_hbm.at[pl.ds(pt0, C)], x_v)
            pltpu.sync_copy(ys_hbm.at[pl.ds(pt0, C)], y_v)
            pltpu.sync_copy(zs_hbm.at[pl.ds(pt0, C)], z_v)

            def vec_body(v, _):
                s = pl.ds(v * L, L)
                tregs = (ticks_v[pl.ds(0, L)], ticks_v[pl.ds(L, L)],
                         ticks_v[pl.ds(2 * L, L)])
                ilx, irx, dlx, drx = axis_calc(x_v[s], tregs)
                ily, iry, dly, dry = axis_calc(y_v[s], tregs)
                ilz, irz, dlz, drz = axis_calc(z_v[s], tregs)
                inv = 1.0 / ((dlx + drx) * (dly + dry) * (dlz + drz))
                rxl = ilx * (TICKS * TICKS)
                rxr = irx * (TICKS * TICKS)
                ryl = ily * TICKS
                ryr = iry * TICKS
                wx = (drx, dlx)
                ry = (ryl, ryr)
                wy = (dry, dly)
                wz = (drz * inv, dlz * inv)
                for a in range(2):
                    rx_a = (rxl, rxr)[a]
                    for b in range(2):
                        ab = a * 2 + b
                        wab = wx[a] * wy[b]
                        idx_v[ab, s] = rx_a + ry[b] + ilz
                        w_v[2 * ab, s] = wab * wz[0]
                        w_v[2 * ab + 1, s] = wab * wz[1]
                return _

            lax.fori_loop(0, vecs, vec_body, None, unroll=False)

            copies = [
                pltpu.async_copy(table_hbm.at[idx_v.at[ab]], rows_v.at[ab], sem)
                for ab in range(4)
            ]
            for cp in copies:
                cp.wait()

            def grp_body(gi, _):
                pb = gi * L
                wvec = [w_v[c, pl.ds(pb, L)] for c in range(8)]
                for p in range(L):
                    pp = pb + p
                    acc = [None] * (F // L)
                    for ab in range(4):
                        for z in range(2):
                            w = wvec[2 * ab + z][p]
                            for j in range(F // L):
                                r = rows_v[ab, pp, pl.ds(z * F + j * L, L)]
                                acc[j] = (r * w if ab == 0 and z == 0
                                          else acc[j] + r * w)
                    for j in range(F // L):
                        out_v[pp, pl.ds(j * L, L)] = acc[j]
                return _

            lax.fori_loop(0, vecs, grp_body, None, unroll=False)
            pltpu.sync_copy(out_v, out_hbm.at[pl.ds(pt0, C)])
            return _

        lax.fori_loop(0, n_chunks, chunk_body, None, unroll=False)

    return body


def kernel(points_to_interp, grid_values):
    n = points_to_interp.shape[0]
    ticks = jnp.arange(-1.0, 1.0, 0.05, dtype=jnp.float32)
    ticks = jnp.pad(ticks, (0, 128 - TICKS))
    xs = points_to_interp[:, 0]
    ys = points_to_interp[:, 1]
    zs = points_to_interp[:, 2]
    tr = jnp.transpose(grid_values, (1, 2, 3, 0)).reshape(
        TICKS * TICKS * TICKS, F)
    table = jnp.concatenate([tr, jnp.roll(tr, -1, axis=0)], axis=1)
    return _interp_sc(n)(ticks, xs, ys, zs, table)


# speedup vs baseline: 58.1679x; 1.3298x over previous
import functools

import jax
import jax.numpy as jnp
from jax import lax
from jax.experimental import pallas as pl
from jax.experimental.pallas import tpu as pltpu
from jax.experimental.pallas import tpu_sc as plsc

F = 64
TICKS = 40
NC, NS, L = 2, 16, 16
NW = NC * NS
C = 64


def _interp_sc(n_points):
    pts_per_w = n_points // NW
    n_chunks = pts_per_w // C
    vecs = C // L

    mesh = plsc.VectorSubcoreMesh(core_axis_name="c", subcore_axis_name="s")

    @functools.partial(
        pl.kernel,
        mesh=mesh,
        out_type=jax.ShapeDtypeStruct((n_points, F), jnp.float32),
        scratch_types=[
            pltpu.VMEM((128,), jnp.float32),
            pltpu.VMEM((2, 3, C), jnp.float32),
            pltpu.VMEM((2, 4, C), jnp.int32),
            pltpu.VMEM((2, 8, C), jnp.float32),
            pltpu.VMEM((2, 4, C, 2 * F), jnp.float32),
            pltpu.VMEM((2, C, F), jnp.float32),
            pltpu.SemaphoreType.DMA,
            pltpu.SemaphoreType.DMA,
            pltpu.SemaphoreType.DMA,
            pltpu.SemaphoreType.DMA,
        ],
    )
    def body(ticks_hbm, pts_hbm, table_hbm, out_hbm,
             ticks_v, pts_v, idx_v, w_v, rows_v, out_v,
             gsem0, gsem1, osem0, osem1):
        wid = lax.axis_index("s") * NC + lax.axis_index("c")
        base = wid * pts_per_w
        gsem = (gsem0, gsem1)
        osem = (osem0, osem1)
        pltpu.sync_copy(ticks_hbm, ticks_v)

        def axis_calc(x, tregs):
            def lookup(k):
                v0 = tregs[0].at[jnp.clip(k, 0, L - 1)].get(
                    mode="promise_in_bounds")
                v1 = tregs[1].at[jnp.clip(k - L, 0, L - 1)].get(
                    mode="promise_in_bounds")
                v2 = tregs[2].at[jnp.clip(k - 2 * L, 0, L - 1)].get(
                    mode="promise_in_bounds")
                return jnp.where(k < L, v0, jnp.where(k < 2 * L, v1, v2))

            est = ((x + 1.0) * 20.0).astype(jnp.int32)
            km1 = est - 1
            kp1 = est + 1

            def contrib(k):
                t = lookup(jnp.clip(k, 0, TICKS - 1))
                c = jnp.where(t < x, 1, 0)
                return jnp.where(k < 0, 1, jnp.where(k > TICKS - 1, 0, c))

            cnt = km1 + contrib(km1) + contrib(est) + contrib(kp1)
            ir = jnp.minimum(cnt, TICKS - 1)
            il = jnp.maximum(ir - 1, 0)
            tl = lookup(il)
            tr = lookup(ir)
            dl = jnp.maximum(x - tl, 0.0)
            dr = jnp.maximum(tr - x, 0.0)
            bz = (dl == 0.0) & (dr == 0.0)
            dl = jnp.where(bz, 1.0, dl)
            dr = jnp.where(bz, 1.0, dr)
            return il, ir, dl, dr

        def compute_and_fire(g, buf):
            cid = wid * n_chunks + g
            pltpu.sync_copy(pts_hbm.at[cid], pts_v.at[buf])

            def vec_body(v, _):
                s = pl.ds(v * L, L)
                tregs = (ticks_v[pl.ds(0, L)], ticks_v[pl.ds(L, L)],
                         ticks_v[pl.ds(2 * L, L)])
                ilx, irx, dlx, drx = axis_calc(pts_v[buf, 0, s], tregs)
                ily, iry, dly, dry = axis_calc(pts_v[buf, 1, s], tregs)
                ilz, irz, dlz, drz = axis_calc(pts_v[buf, 2, s], tregs)
                inv = 1.0 / ((dlx + drx) * (dly + dry) * (dlz + drz))
                rxl = ilx * (TICKS * TICKS)
                rxr = irx * (TICKS * TICKS)
                ryl = ily * TICKS
                ryr = iry * TICKS
                wx = (drx, dlx)
                ry = (ryl, ryr)
                wy = (dry, dly)
                wz = (drz * inv, dlz * inv)
                for a in range(2):
                    rx_a = (rxl, rxr)[a]
                    for b in range(2):
                        ab = a * 2 + b
                        wab = wx[a] * wy[b]
                        idx_v[buf, ab, s] = rx_a + ry[b] + ilz
                        w_v[buf, 2 * ab, s] = wab * wz[0]
                        w_v[buf, 2 * ab + 1, s] = wab * wz[1]
                return _

            lax.fori_loop(0, vecs, vec_body, None, unroll=False)
            for ab in range(4):
                pltpu.async_copy(table_hbm.at[idx_v.at[buf, ab]],
                                 rows_v.at[buf, ab], gsem[buf])

        def wait_combine_store(g, buf):
            pt0 = base + g * C
            for ab in range(4):
                pltpu.make_async_copy(table_hbm.at[idx_v.at[buf, ab]],
                                      rows_v.at[buf, ab], gsem[buf]).wait()

            @pl.when(g >= 2)
            def _():
                pltpu.make_async_copy(
                    out_v.at[buf], out_hbm.at[pl.ds(pt0 - 2 * C, C)],
                    osem[buf]).wait()

            def grp_body(gi, _):
                pb = gi * L
                wvec = [w_v[buf, c, pl.ds(pb, L)] for c in range(8)]
                for p in range(L):
                    pp = pb + p
                    acc = [None] * (F // L)
                    for ab in range(4):
                        for z in range(2):
                            w = wvec[2 * ab + z][p]
                            for j in range(F // L):
                                r = rows_v[buf, ab, pp,
                                           pl.ds(z * F + j * L, L)]
                                acc[j] = (r * w if ab == 0 and z == 0
                                          else acc[j] + r * w)
                    for j in range(F // L):
                        out_v[buf, pp, pl.ds(j * L, L)] = acc[j]
                return _

            lax.fori_loop(0, vecs, grp_body, None, unroll=False)
            pltpu.async_copy(out_v.at[buf], out_hbm.at[pl.ds(pt0, C)],
                             osem[buf])

        compute_and_fire(0, 0)

        def pair_body(i, _):
            g0 = 2 * i
            compute_and_fire(g0 + 1, 1)
            wait_combine_store(g0, 0)

            @pl.when(g0 + 2 < n_chunks)
            def _():
                compute_and_fire(g0 + 2, 0)

            wait_combine_store(g0 + 1, 1)
            return _

        lax.fori_loop(0, n_chunks // 2, pair_body, None, unroll=False)
        end = base + pts_per_w
        pltpu.make_async_copy(out_v.at[0], out_hbm.at[pl.ds(end - 2 * C, C)],
                              osem0).wait()
        pltpu.make_async_copy(out_v.at[1], out_hbm.at[pl.ds(end - C, C)],
                              osem1).wait()

    return body


def kernel(points_to_interp, grid_values):
    n = points_to_interp.shape[0]
    ticks = jnp.arange(-1.0, 1.0, 0.05, dtype=jnp.float32)
    ticks = jnp.pad(ticks, (0, 128 - TICKS))
    pts = points_to_interp.T.reshape(3, n // C, C).transpose(1, 0, 2)
    tr = jnp.transpose(grid_values, (1, 2, 3, 0)).reshape(
        TICKS * TICKS * TICKS, F)
    table = jnp.concatenate([tr, jnp.roll(tr, -1, axis=0)], axis=1)
    return _interp_sc(n)(ticks, pts, table)


# speedup vs baseline: 67.5454x; 1.1612x over previous
import functools

import jax
import jax.numpy as jnp
from jax import lax
from jax.experimental import pallas as pl
from jax.experimental.pallas import tpu as pltpu
from jax.experimental.pallas import tpu_sc as plsc

F = 64
TICKS = 40
NC, NS, L = 2, 16, 16
NW = NC * NS
C = 64


def _interp_sc(n_points):
    pts_per_w = n_points // NW
    n_chunks = pts_per_w // C
    vecs = C // L

    mesh = plsc.VectorSubcoreMesh(core_axis_name="c", subcore_axis_name="s")

    @functools.partial(
        pl.kernel,
        mesh=mesh,
        out_type=jax.ShapeDtypeStruct((n_points, F), jnp.float32),
        scratch_types=[
            pltpu.VMEM((128,), jnp.float32),
            pltpu.VMEM((3, pts_per_w), jnp.float32),
            pltpu.VMEM((2, 4, C), jnp.int32),
            pltpu.VMEM((2, 8, C), jnp.float32),
            pltpu.VMEM((2, 4, C, 2 * F), jnp.float32),
            pltpu.VMEM((2, C, F), jnp.float32),
            pltpu.SemaphoreType.DMA,
            pltpu.SemaphoreType.DMA,
            pltpu.SemaphoreType.DMA,
            pltpu.SemaphoreType.DMA,
        ],
    )
    def body(ticks_hbm, pts_hbm, table_hbm, out_hbm,
             ticks_v, pts_v, idx_v, w_v, rows_v, out_v,
             gsem0, gsem1, osem0, osem1):
        wid = lax.axis_index("s") * NC + lax.axis_index("c")
        base = wid * pts_per_w
        gsem = (gsem0, gsem1)
        osem = (osem0, osem1)
        pltpu.sync_copy(ticks_hbm, ticks_v)
        pltpu.sync_copy(pts_hbm.at[wid], pts_v)

        def axis_calc(x, tregs):
            def lookup(k):
                v0 = tregs[0].at[jnp.clip(k, 0, L - 1)].get(
                    mode="promise_in_bounds")
                v1 = tregs[1].at[jnp.clip(k - L, 0, L - 1)].get(
                    mode="promise_in_bounds")
                v2 = tregs[2].at[jnp.clip(k - 2 * L, 0, L - 1)].get(
                    mode="promise_in_bounds")
                return jnp.where(k < L, v0, jnp.where(k < 2 * L, v1, v2))

            est = ((x + 1.0) * 20.0).astype(jnp.int32)
            km1 = est - 1
            kp1 = est + 1

            def contrib(k):
                t = lookup(jnp.clip(k, 0, TICKS - 1))
                c = jnp.where(t < x, 1, 0)
                return jnp.where(k < 0, 1, jnp.where(k > TICKS - 1, 0, c))

            cnt = km1 + contrib(km1) + contrib(est) + contrib(kp1)
            ir = jnp.minimum(cnt, TICKS - 1)
            il = jnp.maximum(ir - 1, 0)
            tl = lookup(il)
            tr = lookup(ir)
            dl = jnp.maximum(x - tl, 0.0)
            dr = jnp.maximum(tr - x, 0.0)
            bz = (dl == 0.0) & (dr == 0.0)
            dl = jnp.where(bz, 1.0, dl)
            dr = jnp.where(bz, 1.0, dr)
            return il, ir, dl, dr

        def compute_and_fire(g, buf):
            def vec_body(v, _):
                s = pl.ds(v * L, L)
                sp = pl.ds(g * C + v * L, L)
                tregs = (ticks_v[pl.ds(0, L)], ticks_v[pl.ds(L, L)],
                         ticks_v[pl.ds(2 * L, L)])
                ilx, irx, dlx, drx = axis_calc(pts_v[0, sp], tregs)
                ily, iry, dly, dry = axis_calc(pts_v[1, sp], tregs)
                ilz, irz, dlz, drz = axis_calc(pts_v[2, sp], tregs)
                inv = 1.0 / ((dlx + drx) * (dly + dry) * (dlz + drz))
                rxl = ilx * (TICKS * TICKS)
                rxr = irx * (TICKS * TICKS)
                ryl = ily * TICKS
                ryr = iry * TICKS
                wx = (drx, dlx)
                ry = (ryl, ryr)
                wy = (dry, dly)
                wz = (drz * inv, dlz * inv)
                for a in range(2):
                    rx_a = (rxl, rxr)[a]
                    for b in range(2):
                        ab = a * 2 + b
                        wab = wx[a] * wy[b]
                        idx_v[buf, ab, s] = rx_a + ry[b] + ilz
                        w_v[buf, 2 * ab, s] = wab * wz[0]
                        w_v[buf, 2 * ab + 1, s] = wab * wz[1]
                return _

            lax.fori_loop(0, vecs, vec_body, None, unroll=False)
            for ab in range(4):
                pltpu.async_copy(table_hbm.at[idx_v.at[buf, ab]],
                                 rows_v.at[buf, ab], gsem[buf])

        def wait_combine_store(g, buf):
            pt0 = base + g * C
            for ab in range(4):
                pltpu.make_async_copy(table_hbm.at[idx_v.at[buf, ab]],
                                      rows_v.at[buf, ab], gsem[buf]).wait()

            @pl.when(g >= 2)
            def _():
                pltpu.make_async_copy(
                    out_v.at[buf], out_hbm.at[pl.ds(pt0 - 2 * C, C)],
                    osem[buf]).wait()

            def grp_body(gi, _):
                pb = gi * L
                wvec = [w_v[buf, c, pl.ds(pb, L)] for c in range(8)]
                for p in range(L):
                    pp = pb + p
                    acc = [None] * (F // L)
                    for ab in range(4):
                        for z in range(2):
                            w = wvec[2 * ab + z][p]
                            for j in range(F // L):
                                r = rows_v[buf, ab, pp,
                                           pl.ds(z * F + j * L, L)]
                                acc[j] = (r * w if ab == 0 and z == 0
                                          else acc[j] + r * w)
                    for j in range(F // L):
                        out_v[buf, pp, pl.ds(j * L, L)] = acc[j]
                return _

            lax.fori_loop(0, vecs, grp_body, None, unroll=False)
            pltpu.async_copy(out_v.at[buf], out_hbm.at[pl.ds(pt0, C)],
                             osem[buf])

        compute_and_fire(0, 0)

        def pair_body(i, _):
            g0 = 2 * i
            compute_and_fire(g0 + 1, 1)
            wait_combine_store(g0, 0)

            @pl.when(g0 + 2 < n_chunks)
            def _():
                compute_and_fire(g0 + 2, 0)

            wait_combine_store(g0 + 1, 1)
            return _

        lax.fori_loop(0, n_chunks // 2, pair_body, None, unroll=False)
        end = base + pts_per_w
        pltpu.make_async_copy(out_v.at[0], out_hbm.at[pl.ds(end - 2 * C, C)],
                              osem0).wait()
        pltpu.make_async_copy(out_v.at[1], out_hbm.at[pl.ds(end - C, C)],
                              osem1).wait()

    return body


def kernel(points_to_interp, grid_values):
    n = points_to_interp.shape[0]
    ticks = jnp.arange(-1.0, 1.0, 0.05, dtype=jnp.float32)
    ticks = jnp.pad(ticks, (0, 128 - TICKS))
    pts = points_to_interp.T.reshape(3, NW, n // NW).transpose(1, 0, 2)
    tr = jnp.transpose(grid_values, (1, 2, 3, 0)).reshape(
        TICKS * TICKS * TICKS, F)
    table = jnp.concatenate([tr, jnp.roll(tr, -1, axis=0)], axis=1)
    return _interp_sc(n)(ticks, pts, table)
